# baseline (speedup 1.0000x reference)
import math

import jax
import jax.numpy as jnp
from jax.experimental import pallas as pl
from jax.experimental.pallas import tpu as pltpu
from jax.experimental.pallas import tpu_sc as plsc

_THETA = 10000.0
_D = 128
_LN_THETA = math.log(_THETA)
_S = 4096
_R = 128

_NC = 2
_NS = 16
_NW = _NC * _NS
_P_CHUNK = _S // _NW

_R_SC = 16
_R_TC = _R - _R_SC
_BH_BLK = 16
_S_BLK = 512


def _expanded_tables():
    positions = jnp.arange(_S + 1, dtype=jnp.float32)
    exponents = jnp.arange(0, _D, 2, dtype=jnp.float32) / _D
    thetas_k = 1.0 / jnp.power(_THETA, exponents)
    freqs = jnp.outer(positions, thetas_k)
    cos_e = jnp.repeat(jnp.cos(freqs), 2, axis=-1)
    sin_e = jnp.repeat(jnp.sin(freqs), 2, axis=-1)
    sign = jnp.tile(jnp.array([-1.0, 1.0], jnp.float32), _D // 2)
    return cos_e, sin_e * sign


def _sc_body(x_hbm, pos_hbm, cos_hbm, sin_hbm, out_hbm,
             pos_v, cos_v, sin_v, xb0, xb1, ob0, ob1,
             si0, si1, so0, so1, sg):
    wid = jax.lax.axis_index("s") * _NC + jax.lax.axis_index("c")
    base = wid * _P_CHUNK
    sl_hbm = pl.ds(base, _P_CHUNK)

    pltpu.sync_copy(pos_hbm.at[sl_hbm], pos_v)
    pltpu.async_copy(cos_hbm.at[pos_v], cos_v, sg).wait()
    pltpu.async_copy(sin_hbm.at[pos_v], sin_v, sg).wait()

    lane = jax.lax.iota(jnp.int32, 16)
    swap = (lane ^ 1).reshape(16, 1)
    dnums = jax.lax.GatherDimensionNumbers(
        offset_dims=(), collapsed_slice_dims=(0,), start_index_map=(0,))

    def compute(xb, ob):
        def prow(p, c):
            for v in range(_D // 16):
                sl = pl.ds(v * 16, 16)
                xv = xb[p, sl]
                xs = jax.lax.gather(
                    xv, swap, dnums, (1,),
                    mode=jax.lax.GatherScatterMode.PROMISE_IN_BOUNDS)
                ob[p, sl] = xv * cos_v[p, sl] + xs * sin_v[p, sl]
            return c
        jax.lax.fori_loop(0, _P_CHUNK, prow, 0)

    def wait_in(xb, sem):
        pltpu.make_async_copy(x_hbm.at[0, sl_hbm], xb, sem).wait()

    def wait_out(ob, sem):
        pltpu.make_async_copy(ob, out_hbm.at[0, sl_hbm], sem).wait()

    pltpu.async_copy(x_hbm.at[_R_TC + 0, sl_hbm], xb0, si0)
    pltpu.async_copy(x_hbm.at[_R_TC + 1, sl_hbm], xb1, si1)

    def step(t, c):
        r0 = t * 2
        r1 = r0 + 1

        wait_in(xb0, si0)

        @pl.when(t > 0)
        def _():
            wait_out(ob0, so0)

        compute(xb0, ob0)
        pltpu.async_copy(ob0, out_hbm.at[r0, sl_hbm], so0)

        @pl.when(r0 + 2 < _R_SC)
        def _():
            pltpu.async_copy(x_hbm.at[_R_TC + r0 + 2, sl_hbm], xb0, si0)

        wait_in(xb1, si1)

        @pl.when(t > 0)
        def _():
            wait_out(ob1, so1)

        compute(xb1, ob1)
        pltpu.async_copy(ob1, out_hbm.at[r1, sl_hbm], so1)

        @pl.when(r1 + 2 < _R_SC)
        def _():
            pltpu.async_copy(x_hbm.at[_R_TC + r1 + 2, sl_hbm], xb1, si1)

        return c

    jax.lax.fori_loop(0, _R_SC // 2, step, 0)
    wait_out(ob0, so0)
    wait_out(ob1, so1)


def _tc_kernel(pos_ref, x_ref, o_ref, cos_ref, sin_ref):
    lane = jax.lax.broadcasted_iota(jnp.int32, (_S_BLK, _D), 1)

    @pl.when(pl.program_id(1) == 0)
    def _():
        pos = pos_ref[0].astype(jnp.float32)
        pair = (lane // 2).astype(jnp.float32)
        inv_theta = jnp.exp(pair * (-2.0 * _LN_THETA / _D))
        freqs = pos * inv_theta
        sign = jnp.where(lane % 2 == 0, -1.0, 1.0)
        cos_ref[...] = jnp.cos(freqs)
        sin_ref[...] = jnp.sin(freqs) * sign

    x = x_ref[...]
    idx = jax.lax.broadcasted_iota(jnp.int32, x.shape, 2) ^ 1
    x_sw = jnp.take_along_axis(x, idx, axis=2)
    o_ref[...] = x * cos_ref[...][None] + x_sw * sin_ref[...][None]


def kernel(x, token_positions):
    b, h, s, d = x.shape
    bh = b * h
    xr = x.reshape(bh, s, d)
    pos3 = token_positions.reshape(s // _S_BLK, _S_BLK, 1)
    cos_e, sin_m = _expanded_tables()

    sc_run = pl.kernel(
        _sc_body,
        out_type=jax.ShapeDtypeStruct((_R_SC, s, d), jnp.float32),
        mesh=plsc.VectorSubcoreMesh(core_axis_name="c", subcore_axis_name="s"),
        scratch_types=[
            pltpu.VMEM((_P_CHUNK,), jnp.int32),
            pltpu.VMEM((_P_CHUNK, _D), jnp.float32),
            pltpu.VMEM((_P_CHUNK, _D), jnp.float32),
            pltpu.VMEM((_P_CHUNK, _D), jnp.float32),
            pltpu.VMEM((_P_CHUNK, _D), jnp.float32),
            pltpu.VMEM((_P_CHUNK, _D), jnp.float32),
            pltpu.VMEM((_P_CHUNK, _D), jnp.float32),
            pltpu.SemaphoreType.DMA,
            pltpu.SemaphoreType.DMA,
            pltpu.SemaphoreType.DMA,
            pltpu.SemaphoreType.DMA,
            pltpu.SemaphoreType.DMA,
        ],
    )
    out_sc = sc_run(xr, token_positions, cos_e, sin_m)

    out_tc = pl.pallas_call(
        _tc_kernel,
        grid=(s // _S_BLK, _R_TC // _BH_BLK),
        in_specs=[
            pl.BlockSpec((1, _S_BLK, 1), lambda i, j: (i, 0, 0)),
            pl.BlockSpec((_BH_BLK, _S_BLK, d), lambda i, j: (j, i, 0)),
        ],
        out_specs=pl.BlockSpec((_BH_BLK, _S_BLK, d), lambda i, j: (j, i, 0)),
        out_shape=jax.ShapeDtypeStruct((bh, s, d), x.dtype),
        scratch_shapes=[
            pltpu.VMEM((_S_BLK, _D), jnp.float32),
            pltpu.VMEM((_S_BLK, _D), jnp.float32),
        ],
        compiler_params=pltpu.CompilerParams(
            dimension_semantics=("parallel", "arbitrary"),
        ),
    )(pos3, xr)

    out = jax.lax.dynamic_update_slice(out_tc, out_sc, (_R_TC, 0, 0))
    return out.reshape(b, h, s, d)


# speedup vs baseline: 1.0030x; 1.0030x over previous
import math

import jax
import jax.numpy as jnp
from jax.experimental import pallas as pl
from jax.experimental.pallas import tpu as pltpu
from jax.experimental.pallas import tpu_sc as plsc

_THETA = 10000.0
_D = 128
_LN_THETA = math.log(_THETA)
_S = 4096
_R = 128

_NC = 2
_NS = 16
_NW = _NC * _NS
_P_CHUNK = _S // _NW

_R_SC = 16
_R_TC = _R - _R_SC
_BH_BLK = 16
_S_BLK = 512


def _expanded_tables():
    positions = jnp.arange(_S + 1, dtype=jnp.float32)
    exponents = jnp.arange(0, _D, 2, dtype=jnp.float32) / _D
    thetas_k = 1.0 / jnp.power(_THETA, exponents)
    freqs = jnp.outer(positions, thetas_k)
    cos_e = jnp.repeat(jnp.cos(freqs), 2, axis=-1)
    sin_e = jnp.repeat(jnp.sin(freqs), 2, axis=-1)
    sign = jnp.tile(jnp.array([-1.0, 1.0], jnp.float32), _D // 2)
    return cos_e, sin_e * sign


def _sc_body(x_hbm, pos_hbm, cos_hbm, sin_hbm, out_hbm,
             pos_v, cos_v, sin_v, xb0, xb1, ob0, ob1,
             si0, si1, so0, so1, sg):
    wid = jax.lax.axis_index("s") * _NC + jax.lax.axis_index("c")
    base = wid * _P_CHUNK
    sl_hbm = pl.ds(base, _P_CHUNK)

    pltpu.sync_copy(pos_hbm.at[sl_hbm], pos_v)
    pltpu.async_copy(cos_hbm.at[pos_v], cos_v, sg).wait()
    pltpu.async_copy(sin_hbm.at[pos_v], sin_v, sg).wait()

    lane = jax.lax.iota(jnp.int32, 16)
    swap = (lane ^ 1).reshape(16, 1)
    dnums = jax.lax.GatherDimensionNumbers(
        offset_dims=(), collapsed_slice_dims=(0,), start_index_map=(0,))

    def compute(xb, ob):
        def prow(p, c):
            for v in range(_D // 16):
                sl = pl.ds(v * 16, 16)
                xv = xb[p, sl]
                xs = jax.lax.gather(
                    xv, swap, dnums, (1,),
                    mode=jax.lax.GatherScatterMode.PROMISE_IN_BOUNDS)
                ob[p, sl] = xv * cos_v[p, sl] + xs * sin_v[p, sl]
            return c
        jax.lax.fori_loop(0, _P_CHUNK, prow, 0)

    def wait_in(xb, sem):
        pltpu.make_async_copy(x_hbm.at[0, sl_hbm], xb, sem).wait()

    def wait_out(ob, sem):
        pltpu.make_async_copy(ob, out_hbm.at[0, sl_hbm], sem).wait()

    pltpu.async_copy(x_hbm.at[_R_TC + 0, sl_hbm], xb0, si0)
    pltpu.async_copy(x_hbm.at[_R_TC + 1, sl_hbm], xb1, si1)

    def step(t, c):
        r0 = t * 2
        r1 = r0 + 1

        wait_in(xb0, si0)

        @pl.when(t > 0)
        def _():
            wait_out(ob0, so0)

        compute(xb0, ob0)
        pltpu.async_copy(ob0, out_hbm.at[r0, sl_hbm], so0)

        @pl.when(r0 + 2 < _R_SC)
        def _():
            pltpu.async_copy(x_hbm.at[_R_TC + r0 + 2, sl_hbm], xb0, si0)

        wait_in(xb1, si1)

        @pl.when(t > 0)
        def _():
            wait_out(ob1, so1)

        compute(xb1, ob1)
        pltpu.async_copy(ob1, out_hbm.at[r1, sl_hbm], so1)

        @pl.when(r1 + 2 < _R_SC)
        def _():
            pltpu.async_copy(x_hbm.at[_R_TC + r1 + 2, sl_hbm], xb1, si1)

        return c

    jax.lax.fori_loop(0, _R_SC // 2, step, 0)
    wait_out(ob0, so0)
    wait_out(ob1, so1)


def _tc_kernel(pos_ref, x_ref, o_ref, cos_ref, sin_ref):
    lane = jax.lax.broadcasted_iota(jnp.int32, (_S_BLK, _D), 1)

    @pl.when(pl.program_id(1) == 0)
    def _():
        pos = pos_ref[0].astype(jnp.float32)
        pair = (lane // 2).astype(jnp.float32)
        inv_theta = jnp.exp(pair * (-2.0 * _LN_THETA / _D))
        freqs = pos * inv_theta
        sign = jnp.where(lane % 2 == 0, -1.0, 1.0)
        cos_ref[...] = jnp.cos(freqs)
        sin_ref[...] = jnp.sin(freqs) * sign

    x = x_ref[...]
    idx = jax.lax.broadcasted_iota(jnp.int32, x.shape, 2) ^ 1
    x_sw = jnp.take_along_axis(x, idx, axis=2)
    o_ref[...] = x * cos_ref[...][None] + x_sw * sin_ref[...][None]


def kernel(x, token_positions):
    b, h, s, d = x.shape
    bh = b * h
    xr = x.reshape(bh, s, d)
    pos3 = token_positions.reshape(s // _S_BLK, _S_BLK, 1)
    cos_e, sin_m = _expanded_tables()

    sc_run = pl.kernel(
        _sc_body,
        out_type=jax.ShapeDtypeStruct((_R_SC, s, d), jnp.float32),
        mesh=plsc.VectorSubcoreMesh(core_axis_name="c", subcore_axis_name="s"),
        scratch_types=[
            pltpu.VMEM((_P_CHUNK,), jnp.int32),
            pltpu.VMEM((_P_CHUNK, _D), jnp.float32),
            pltpu.VMEM((_P_CHUNK, _D), jnp.float32),
            pltpu.VMEM((_P_CHUNK, _D), jnp.float32),
            pltpu.VMEM((_P_CHUNK, _D), jnp.float32),
            pltpu.VMEM((_P_CHUNK, _D), jnp.float32),
            pltpu.VMEM((_P_CHUNK, _D), jnp.float32),
            pltpu.SemaphoreType.DMA,
            pltpu.SemaphoreType.DMA,
            pltpu.SemaphoreType.DMA,
            pltpu.SemaphoreType.DMA,
            pltpu.SemaphoreType.DMA,
        ],
        cost_estimate=pl.CostEstimate(
            flops=4 * _R_SC * _S * _D,
            bytes_accessed=2 * _R_SC * _S * _D * 4,
            transcendentals=0,
        ),
    )
    out_sc = sc_run(xr, token_positions, cos_e, sin_m)

    out_tc = pl.pallas_call(
        _tc_kernel,
        grid=(s // _S_BLK, _R_TC // _BH_BLK),
        in_specs=[
            pl.BlockSpec((1, _S_BLK, 1), lambda i, j: (i, 0, 0)),
            pl.BlockSpec((_BH_BLK, _S_BLK, d), lambda i, j: (j, i, 0)),
        ],
        out_specs=pl.BlockSpec((_BH_BLK, _S_BLK, d), lambda i, j: (j, i, 0)),
        out_shape=jax.ShapeDtypeStruct((bh, s, d), x.dtype),
        scratch_shapes=[
            pltpu.VMEM((_S_BLK, _D), jnp.float32),
            pltpu.VMEM((_S_BLK, _D), jnp.float32),
        ],
        compiler_params=pltpu.CompilerParams(
            dimension_semantics=("parallel", "arbitrary"),
        ),
    )(pos3, xr)

    out = jax.lax.dynamic_update_slice(out_tc, out_sc, (_R_TC, 0, 0))
    return out.reshape(b, h, s, d)


# speedup vs baseline: 1.1485x; 1.1451x over previous
import jax
import jax.numpy as jnp
from jax.experimental import pallas as pl
from jax.experimental.pallas import tpu as pltpu
from jax.experimental.pallas import tpu_sc as plsc

_THETA = 10000.0
_D = 128
_NC = 2
_NS = 16
_NW = _NC * _NS
_S = 4096
_P_CHUNK = _S // _NW

_BH_BLK = 32
_S_BLK = 512


def _expanded_tables():
    positions = jnp.arange(_S + 1, dtype=jnp.float32)
    exponents = jnp.arange(0, _D, 2, dtype=jnp.float32) / _D
    thetas_k = 1.0 / jnp.power(_THETA, exponents)
    freqs = jnp.outer(positions, thetas_k)
    cos_e = jnp.repeat(jnp.cos(freqs), 2, axis=-1)
    sin_e = jnp.repeat(jnp.sin(freqs), 2, axis=-1)
    sign = jnp.tile(jnp.array([-1.0, 1.0], jnp.float32), _D // 2)
    return cos_e, sin_e * sign


def _sc_gather_body(pos_hbm, cos_hbm, sin_hbm, outc_hbm, outs_hbm,
                    pos_v, cos_v, sin_v, sg_c, sg_s):
    wid = jax.lax.axis_index("s") * _NC + jax.lax.axis_index("c")
    sl = pl.ds(wid * _P_CHUNK, _P_CHUNK)
    pltpu.sync_copy(pos_hbm.at[sl], pos_v)
    c1 = pltpu.async_copy(cos_hbm.at[pos_v], cos_v, sg_c)
    c2 = pltpu.async_copy(sin_hbm.at[pos_v], sin_v, sg_s)
    c1.wait()
    c3 = pltpu.async_copy(cos_v, outc_hbm.at[sl], sg_c)
    c2.wait()
    c4 = pltpu.async_copy(sin_v, outs_hbm.at[sl], sg_s)
    c3.wait()
    c4.wait()


def _rope_tc_kernel(cos_ref, sin_ref, x_ref, o_ref):
    x = x_ref[...]
    idx = jax.lax.broadcasted_iota(jnp.int32, x.shape, 2) ^ 1
    x_sw = jnp.take_along_axis(x, idx, axis=2)
    o_ref[...] = x * cos_ref[...][None] + x_sw * sin_ref[...][None]


def kernel(x, token_positions):
    b, h, s, d = x.shape
    bh = b * h
    xr = x.reshape(bh, s, d)
    cos_e, sin_m = _expanded_tables()

    gather = pl.kernel(
        _sc_gather_body,
        out_type=(
            jax.ShapeDtypeStruct((s, d), jnp.float32),
            jax.ShapeDtypeStruct((s, d), jnp.float32),
        ),
        mesh=plsc.VectorSubcoreMesh(core_axis_name="c", subcore_axis_name="s"),
        scratch_types=[
            pltpu.VMEM((_P_CHUNK,), jnp.int32),
            pltpu.VMEM((_P_CHUNK, _D), jnp.float32),
            pltpu.VMEM((_P_CHUNK, _D), jnp.float32),
            pltpu.SemaphoreType.DMA,
            pltpu.SemaphoreType.DMA,
        ],
    )
    cos_g, sin_g = gather(token_positions, cos_e, sin_m)

    out = pl.pallas_call(
        _rope_tc_kernel,
        grid=(s // _S_BLK, bh // _BH_BLK),
        in_specs=[
            pl.BlockSpec((_S_BLK, d), lambda i, j: (i, 0)),
            pl.BlockSpec((_S_BLK, d), lambda i, j: (i, 0)),
            pl.BlockSpec((_BH_BLK, _S_BLK, d), lambda i, j: (j, i, 0)),
        ],
        out_specs=pl.BlockSpec((_BH_BLK, _S_BLK, d), lambda i, j: (j, i, 0)),
        out_shape=jax.ShapeDtypeStruct((bh, s, d), x.dtype),
        compiler_params=pltpu.CompilerParams(
            dimension_semantics=("parallel", "arbitrary"),
        ),
    )(cos_g, sin_g, xr)
    return out.reshape(b, h, s, d)
